# baseline (speedup 1.0000x reference)
import jax
import jax.numpy as jnp
from jax import lax
from jax.experimental import pallas as pl
from jax.experimental.pallas import tpu as pltpu

_MAXPOS = 31
_NBIN = 63
_MIB = 1024 * 1024


def _msa_body(msa_ref, w_ref, q_ref, o_ref):
    tn, L, d_in = msa_ref.shape[1:]
    x = msa_ref[0].reshape(tn * L, d_in).astype(jnp.bfloat16)
    y = jnp.dot(x, w_ref[...], preferred_element_type=jnp.float32)
    o_ref[0] = y.reshape(tn, L, -1) + q_ref[...]


def _msa_embed(msa, w_bf, q, tn):
    B, N, L, d_in = msa.shape
    d_msa = w_bf.shape[1]
    return pl.pallas_call(
        _msa_body,
        out_shape=jax.ShapeDtypeStruct((B, N, L, d_msa), jnp.float32),
        grid=(B, N // tn),
        in_specs=[
            pl.BlockSpec((1, tn, L, d_in), lambda b, n: (b, n, 0, 0)),
            pl.BlockSpec((d_in, d_msa), lambda b, n: (0, 0)),
            pl.BlockSpec((1, L, d_msa), lambda b, n: (b, 0, 0)),
        ],
        out_specs=pl.BlockSpec((1, tn, L, d_msa), lambda b, n: (b, n, 0, 0)),
        compiler_params=pltpu.CompilerParams(
            dimension_semantics=("parallel", "parallel"),
            vmem_limit_bytes=56 * _MIB),
    )(msa, w_bf, q)


def _pair_body(ii_ref, ij_ref, sc_ref, left_ref, right_ref, tab_ref, o_ref):
    ti, L = sc_ref.shape[1:]
    nb, dp = tab_ref.shape
    sep = jnp.clip(ij_ref[0] - ii_ref[0] + _MAXPOS, 0, _NBIN - 1)
    sep = jnp.where(sc_ref[0] == 1, sep, _NBIN)
    oh = (lax.broadcasted_iota(jnp.int32, (ti, L, nb), 2)
          == sep[:, :, None]).astype(jnp.bfloat16)
    pos = jnp.dot(oh.reshape(ti * L, nb), tab_ref[...],
                  preferred_element_type=jnp.float32)
    o_ref[0] = (pos.reshape(ti, L, dp)
                + left_ref[0][None, :, :] + right_ref[0][:, None, :])


def _pair_embed(idx32, sc32, left, right, tab_bf, ti):
    B, L, dp = left.shape
    nb = tab_bf.shape[0]
    return pl.pallas_call(
        _pair_body,
        out_shape=jax.ShapeDtypeStruct((B, L, L, dp), jnp.float32),
        grid=(B, L // ti),
        in_specs=[
            pl.BlockSpec((1, ti, 1), lambda b, i: (b, i, 0)),
            pl.BlockSpec((1, 1, L), lambda b, i: (b, 0, 0)),
            pl.BlockSpec((1, ti, L), lambda b, i: (b, i, 0)),
            pl.BlockSpec((1, L, dp), lambda b, i: (b, 0, 0)),
            pl.BlockSpec((1, ti, dp), lambda b, i: (b, i, 0)),
            pl.BlockSpec((nb, dp), lambda b, i: (0, 0)),
        ],
        out_specs=pl.BlockSpec((1, ti, L, dp), lambda b, i: (b, i, 0, 0)),
        compiler_params=pltpu.CompilerParams(
            dimension_semantics=("parallel", "parallel"),
            vmem_limit_bytes=56 * _MIB),
    )(idx32[:, :, None], idx32[:, None, :], sc32, left, right, tab_bf)


def _pick_tile(n, want, step):
    if n % want == 0:
        return want
    for t in range(want - step, 0, -step):
        if n % t == 0:
            return t
    return n


def kernel(emb_w, emb_b, emb_q, emb_left, emb_right, pos_emb, pos_emb_chain,
           pos_proj_w, msa, seq, idx, same_chain):
    B, N, L, _ = msa.shape
    d_pair = emb_left.shape[1]
    nb = pos_emb.shape[0]

    seq32 = seq.astype(jnp.int32)
    q = emb_q[seq32] + emb_b
    left = emb_left[seq32]
    right = emb_right[seq32]

    tab = pos_emb @ pos_proj_w[:d_pair] + pos_emb_chain[1] @ pos_proj_w[d_pair:]
    tab = tab.at[nb - 1].add((pos_emb_chain[0] - pos_emb_chain[1])
                             @ pos_proj_w[d_pair:])

    msa_out = _msa_embed(msa, emb_w.astype(jnp.bfloat16), q,
                         _pick_tile(N, 16, 1))
    pair_out = _pair_embed(idx.astype(jnp.int32), same_chain.astype(jnp.int32),
                           left, right, tab.astype(jnp.bfloat16),
                           _pick_tile(L, 48, 8))
    return msa_out, pair_out
